# baseline (speedup 1.0000x reference)
import functools

import jax
import jax.numpy as jnp
from jax import lax
from jax.experimental import pallas as pl
from jax.experimental.pallas import tpu as pltpu
from jax.experimental.pallas import tpu_sc as plsc

NUM_NODES = 1000000
DIM = 64
BATCH = 16384
NC, NS, L = 2, 16, 16
NW = NC * NS
BPW = BATCH // NW
CHUNK = 128
NCHUNK = BPW // CHUNK

_mesh = plsc.VectorSubcoreMesh(
    core_axis_name="c", subcore_axis_name="s", num_cores=NC, num_subcores=NS)


@functools.partial(
    pl.kernel,
    out_type=jax.ShapeDtypeStruct((BATCH,), jnp.float32),
    mesh=_mesh,
    scratch_types=[
        pltpu.VMEM((NCHUNK, CHUNK), jnp.int32),
        pltpu.VMEM((NCHUNK, CHUNK), jnp.int32),
        pltpu.VMEM((BPW, DIM), jnp.float32),
        pltpu.VMEM((BPW, DIM), jnp.float32),
        pltpu.VMEM((BPW,), jnp.float32),
        pltpu.VMEM((L, L + 1), jnp.float32),
        pltpu.SemaphoreType.DMA,
    ],
    compiler_params=pltpu.CompilerParams(
        needs_layout_passes=False, use_tc_tiling_on_sc=False),
)
def _sc_dot(u_hbm, v_hbm, emb_hbm, ctx_hbm, out_hbm,
            iu_v, iv_v, e_v, c_v, prod_v, tr_v, sem):
    wid = lax.axis_index("s") * NC + lax.axis_index("c")
    row0 = wid * NCHUNK
    pltpu.sync_copy(u_hbm.at[pl.ds(row0, NCHUNK)], iu_v)
    pltpu.sync_copy(v_hbm.at[pl.ds(row0, NCHUNK)], iv_v)
    copies = []
    for j in range(NCHUNK):
        copies.append(pltpu.async_copy(
            emb_hbm.at[iu_v.at[j]], e_v.at[pl.ds(j * CHUNK, CHUNK)], sem))
        copies.append(pltpu.async_copy(
            ctx_hbm.at[iv_v.at[j]], c_v.at[pl.ds(j * CHUNK, CHUNK)], sem))
    for cp in copies:
        cp.wait()

    lane = lax.iota(jnp.int32, L)
    col_idx = [lane * 0 + j for j in range(L)]

    def grp_body(g, _):
        for i in range(L):
            r = g * L + i
            acc = e_v[r, pl.ds(0, L)] * c_v[r, pl.ds(0, L)]
            for k in range(1, DIM // L):
                acc = acc + e_v[r, pl.ds(k * L, L)] * c_v[r, pl.ds(k * L, L)]
            tr_v[i, pl.ds(0, L)] = acc
        tot = plsc.load_gather(tr_v, [lane, col_idx[0]])
        for j in range(1, L):
            tot = tot + plsc.load_gather(tr_v, [lane, col_idx[j]])
        prod_v[pl.ds(g * L, L)] = tot
        return 0

    lax.fori_loop(0, BPW // L, grp_body, 0)
    pltpu.sync_copy(prod_v, out_hbm.at[pl.ds(wid * BPW, BPW)])


def _loss_body(sign_ref, prod_ref, out_ref):
    x = sign_ref[...] * prod_ref[...]
    ls = jnp.minimum(x, 0.0) - jnp.log(1.0 + jnp.exp(-jnp.abs(x)))
    out_ref[...] = jnp.reshape(-jnp.sum(ls), (1, 1))


_loss = pl.pallas_call(
    _loss_body,
    out_shape=jax.ShapeDtypeStruct((1, 1), jnp.float32),
)


def kernel(u, v, sign, emb_table, ctx_table):
    u2 = u.reshape(BATCH // CHUNK, CHUNK)
    v2 = v.reshape(BATCH // CHUNK, CHUNK)
    prod = _sc_dot(u2, v2, emb_table, ctx_table)
    loss = _loss(sign.reshape(CHUNK, CHUNK), prod.reshape(CHUNK, CHUNK))
    return loss[0, 0]
